# baseline (speedup 1.0000x reference)
import jax
import jax.numpy as jnp
from jax import lax
from jax.experimental import pallas as pl
from jax.experimental.pallas import tpu as pltpu
from jax.experimental.pallas import tpu_sc as plsc

BATCH = 16384
HIST = 50
EMBEDDING_DIM = 64
NROWS = 1000000

_INFO = plsc.get_sparse_core_info()
_NC = _INFO.num_cores
_NS = _INFO.num_subcores
_NW = _NC * _NS

_B = BATCH * HIST
_BLK = 128
_NBLOCKS = _B // _BLK
_BLOCKS_PER_WORKER = _NBLOCKS // _NW
_TPB = BATCH // _BLK
_NBUF = 2
_PITCH = _BLK + 1

_ACOLS = 7813
_PADROWS = _ACOLS * 128
_APITCH = 129


def _relayout_body(tab_hbm, out_hbm, in_v, tr_v, in_sems, out_sems):
    wid = lax.axis_index("s") * _NC + lax.axis_index("c")
    iota = jax.lax.iota(jnp.int32, 16)

    def blk_of(i):
        return wid + _NW * i

    def fire_in(i, b):
        blk = blk_of(i)
        c0 = pl.multiple_of(blk * 128, 128)
        pltpu.async_copy(
            tab_hbm.at[:, pl.ds(c0, 128)], in_v.at[b], in_sems[b]
        )

    def wait_in(i, b):
        pltpu.make_async_copy(
            tab_hbm.at[:, pl.ds(pl.multiple_of(blk_of(i) * 128, 128), 128)],
            in_v.at[b],
            in_sems[b],
        ).wait()

    def fire_out(i, b):
        blk = blk_of(i)
        r0 = pl.multiple_of(blk * 128, 128)
        pltpu.async_copy(
            tr_v.at[b, :, pl.ds(0, 128)],
            out_hbm.at[pl.ds(r0, 128)],
            out_sems[b],
        )

    def wait_out(i, b):
        blk = blk_of(i)
        r0 = pl.multiple_of(blk * 128, 128)
        pltpu.make_async_copy(
            tr_v.at[b, :, pl.ds(0, 128)],
            out_hbm.at[pl.ds(r0, 128)],
            out_sems[b],
        ).wait()

    row_vecs = [iota + 16 * m for m in range(8)]

    def transpose(b):
        @plsc.parallel_loop(0, 64, unroll=4)
        def _dim(d):
            dvec = jnp.full((16,), d, dtype=jnp.int32)
            bvec = jnp.full((16,), b, dtype=jnp.int32)
            for m in range(8):
                vals = in_v[b, d, pl.ds(16 * m, 16)]
                plsc.store_scatter(tr_v, [bvec, row_vecs[m], dvec], vals)

    n_i = (_ACOLS - wid + _NW - 1) // _NW

    for b in range(_NBUF):
        @pl.when(b < n_i)
        def _():
            fire_in(b, b)

    def step(i, b):
        wait_in(i, b)

        @pl.when(i >= _NBUF)
        def _():
            wait_out(i - _NBUF, b)

        transpose(b)
        fire_out(i, b)

        @pl.when(i + _NBUF < n_i)
        def _():
            fire_in(i + _NBUF, b)

    @pl.loop(0, 256, step=_NBUF)
    def _blk(i0):
        for b in range(_NBUF):
            i = i0 + b

            @pl.when(i < n_i)
            def _():
                step(i, b)

    for b in range(_NBUF):
        @pl.when(n_i - _NBUF + b >= 0)
        def _():
            wait_out(n_i - _NBUF + b, b)


def _gather_body(idx_hbm, table_hbm, out_hbm, idx_v, raw_v, tile_v, gat_sems, out_sems):
    wid = lax.axis_index("s") * _NC + lax.axis_index("c")
    blk0 = wid * _BLOCKS_PER_WORKER
    base_irow = pl.multiple_of(blk0, 8)

    pltpu.sync_copy(idx_hbm.at[pl.ds(base_irow, _BLOCKS_PER_WORKER)], idx_v)

    iota = jax.lax.iota(jnp.int32, 16)

    def fire_gather(g, b):
        pltpu.async_copy(table_hbm.at[idx_v.at[g]], raw_v.at[b], gat_sems[b])

    def wait_gather(g, b):
        pltpu.make_async_copy(
            table_hbm.at[idx_v.at[g]], raw_v.at[b], gat_sems[b]
        ).wait()

    def out_dst(g):
        blk = blk0 + g
        h = blk // _TPB
        t = blk % _TPB
        return out_hbm.at[h, :, t]

    def fire_out(g, b):
        pltpu.async_copy(
            tile_v.at[b, :, :, pl.ds(0, _BLK)], out_dst(g), out_sems[b]
        )

    def wait_out(g, b):
        pltpu.make_async_copy(
            tile_v.at[b, :, :, pl.ds(0, _BLK)], out_dst(g), out_sems[b]
        ).wait()

    gg_vecs = [(16 * j + iota) // 8 for j in range(4)]
    dd_vecs = [(16 * j + iota) % 8 for j in range(4)]

    def transpose(b):
        bvec = jnp.full((16,), b, dtype=jnp.int32)

        @plsc.parallel_loop(0, _BLK, unroll=4)
        def _tok(c):
            cvec = jnp.full((16,), c, dtype=jnp.int32)
            for j in range(4):
                vals = raw_v[b, c, pl.ds(16 * j, 16)]
                plsc.store_scatter(
                    tile_v, [bvec, gg_vecs[j], dd_vecs[j], cvec], vals
                )

    for b in range(_NBUF):
        fire_gather(b, b)

    @pl.loop(0, _BLOCKS_PER_WORKER, step=_NBUF)
    def _blk(g0):
        for b in range(_NBUF):
            g = g0 + b
            wait_gather(g, b)

            @pl.when(g0 > 0)
            def _():
                wait_out(g - _NBUF, b)

            transpose(b)
            fire_out(g, b)

            @pl.when(g + _NBUF < _BLOCKS_PER_WORKER)
            def _():
                fire_gather(g + _NBUF, b)

    for b in range(_NBUF):
        wait_out(_BLOCKS_PER_WORKER - _NBUF + b, b)


@jax.jit
def _relayout(tabT):
    mesh = plsc.VectorSubcoreMesh(core_axis_name="c", subcore_axis_name="s")
    fn = pl.kernel(
        _relayout_body,
        out_type=jax.ShapeDtypeStruct((_PADROWS, 128), jnp.float32),
        mesh=mesh,
        scratch_types=[
            pltpu.VMEM((_NBUF, 64, 128), jnp.float32),
            pltpu.VMEM((_NBUF, 128, _APITCH), jnp.float32),
            [pltpu.SemaphoreType.DMA] * _NBUF,
            [pltpu.SemaphoreType.DMA] * _NBUF,
        ],
        compiler_params=pltpu.CompilerParams(
            use_tc_tiling_on_sc=True,
            disable_bounds_checks=True,
            needs_layout_passes=False,
        ),
    )
    return fn(tabT)


@jax.jit
def _lookup(idx2d, table2):
    mesh = plsc.VectorSubcoreMesh(core_axis_name="c", subcore_axis_name="s")
    fn = pl.kernel(
        _gather_body,
        out_type=jax.ShapeDtypeStruct((HIST, 8, _TPB, 8, _BLK), jnp.float32),
        mesh=mesh,
        scratch_types=[
            pltpu.VMEM((_BLOCKS_PER_WORKER, _BLK), jnp.int32),
            pltpu.VMEM((_NBUF, _BLK, 128), jnp.float32),
            pltpu.VMEM((_NBUF, 8, 8, _PITCH), jnp.float32),
            [pltpu.SemaphoreType.DMA] * _NBUF,
            [pltpu.SemaphoreType.DMA] * _NBUF,
        ],
        compiler_params=pltpu.CompilerParams(
            use_tc_tiling_on_sc=False, needs_layout_passes=False
        ),
    )
    return fn(idx2d, table2)


def kernel(token_ids, embedding):
    idx2d = jnp.reshape(token_ids.T.astype(jnp.int32), (_NBLOCKS, _BLK))
    table2 = _relayout(embedding.T)
    out5 = _lookup(idx2d, table2)
    return out5.transpose(2, 4, 0, 1, 3).reshape(BATCH, HIST, EMBEDDING_DIM)


# speedup vs baseline: 1.4224x; 1.4224x over previous
import jax
import jax.numpy as jnp
from jax import lax
from jax.experimental import pallas as pl
from jax.experimental.pallas import tpu as pltpu
from jax.experimental.pallas import tpu_sc as plsc

BATCH = 16384
HIST = 50
EMBEDDING_DIM = 64

_INFO = plsc.get_sparse_core_info()
_NC = _INFO.num_cores
_NS = _INFO.num_subcores
_NW = _NC * _NS

_B = BATCH * HIST
_BLK = 128
_NBLOCKS = _B // _BLK
_BLOCKS_PER_WORKER = _NBLOCKS // _NW
_TPB = BATCH // _BLK
_NBUF = 2
_PITCH = _BLK + 1


def _body(idx_hbm, table_hbm, out_hbm, idx_v, raw_v, tile_v, gat_sems, out_sems):
    wid = lax.axis_index("s") * _NC + lax.axis_index("c")
    blk0 = wid * _BLOCKS_PER_WORKER
    base_irow = pl.multiple_of(blk0, 8)

    pltpu.sync_copy(idx_hbm.at[pl.ds(base_irow, _BLOCKS_PER_WORKER)], idx_v)

    iota = jax.lax.iota(jnp.int32, 16)

    def fire_gather(g, b):
        pltpu.async_copy(table_hbm.at[idx_v.at[g]], raw_v.at[b], gat_sems[b])

    def wait_gather(g, b):
        pltpu.make_async_copy(
            table_hbm.at[idx_v.at[g]], raw_v.at[b], gat_sems[b]
        ).wait()

    def out_dst(g):
        blk = blk0 + g
        h = blk // _TPB
        t = blk % _TPB
        return out_hbm.at[h, :, t]

    def fire_out(g, b):
        pltpu.async_copy(
            tile_v.at[b, :, :, pl.ds(0, _BLK)], out_dst(g), out_sems[b]
        )

    def wait_out(g, b):
        pltpu.make_async_copy(
            tile_v.at[b, :, :, pl.ds(0, _BLK)], out_dst(g), out_sems[b]
        ).wait()

    gg_vecs = [(16 * j + iota) // 8 for j in range(4)]
    dd_vecs = [(16 * j + iota) % 8 for j in range(4)]

    def transpose(b):
        bvec = jnp.full((16,), b, dtype=jnp.int32)

        @plsc.parallel_loop(0, _BLK, unroll=4)
        def _tok(c):
            cvec = jnp.full((16,), c, dtype=jnp.int32)
            for j in range(4):
                vals = raw_v[b, c, pl.ds(16 * j, 16)]
                plsc.store_scatter(
                    tile_v, [bvec, gg_vecs[j], dd_vecs[j], cvec], vals
                )

    for b in range(_NBUF):
        fire_gather(b, b)

    @pl.loop(0, _BLOCKS_PER_WORKER, step=_NBUF)
    def _blk(g0):
        for b in range(_NBUF):
            g = g0 + b
            wait_gather(g, b)

            @pl.when(g0 > 0)
            def _():
                wait_out(g - _NBUF, b)

            transpose(b)
            fire_out(g, b)

            @pl.when(g + _NBUF < _BLOCKS_PER_WORKER)
            def _():
                fire_gather(g + _NBUF, b)

    for b in range(_NBUF):
        wait_out(_BLOCKS_PER_WORKER - _NBUF + b, b)


@jax.jit
def _lookup(idx2d, table):
    mesh = plsc.VectorSubcoreMesh(core_axis_name="c", subcore_axis_name="s")
    fn = pl.kernel(
        _body,
        out_type=jax.ShapeDtypeStruct((HIST, 8, _TPB, 8, _BLK), jnp.float32),
        mesh=mesh,
        scratch_types=[
            pltpu.VMEM((_BLOCKS_PER_WORKER, _BLK), jnp.int32),
            pltpu.VMEM((_NBUF, _BLK, EMBEDDING_DIM), jnp.float32),
            pltpu.VMEM((_NBUF, 8, 8, _PITCH), jnp.float32),
            [pltpu.SemaphoreType.DMA] * _NBUF,
            [pltpu.SemaphoreType.DMA] * _NBUF,
        ],
        compiler_params=pltpu.CompilerParams(
            use_tc_tiling_on_sc=False, needs_layout_passes=False
        ),
    )
    return fn(idx2d, table)


def kernel(token_ids, embedding):
    idx2d = jnp.reshape(token_ids.T.astype(jnp.int32), (_NBLOCKS, _BLK))
    out5 = _lookup(idx2d, embedding)
    return out5.transpose(2, 4, 0, 1, 3).reshape(BATCH, HIST, EMBEDDING_DIM)
